# Initial kernel scaffold; baseline (speedup 1.0000x reference)
import jax
import jax.numpy as jnp
from jax.experimental import pallas as pl


def kernel(Trans_emb, params, node_index1, edge_index1, edge_type1, node_type1, node_mask1, batch1, node_index2, edge_index2, edge_type2, node_type2, node_mask2, batch2, drug_index1, drug_index2):
    raise NotImplementedError("write your pallas kernel here")


# speedup vs baseline: 1.2477x; 1.2477x over previous
import functools
import jax
import jax.numpy as jnp
from jax.experimental import pallas as pl
from jax.experimental.pallas import tpu as pltpu

HD = 128
N_NODES = 10000
B_GRAPHS = 64


def _mm_body(a_ref, b_ref, o_ref, *, act):
    @pl.when(pl.program_id(1) == 0)
    def _():
        o_ref[...] = jnp.zeros_like(o_ref)

    o_ref[...] += jnp.dot(a_ref[...], b_ref[...],
                          preferred_element_type=jnp.float32)
    if act == 'relu':
        @pl.when(pl.program_id(1) == pl.num_programs(1) - 1)
        def _():
            o_ref[...] = jnp.maximum(o_ref[...], 0.0)


def _mm(a, b, act=None):
    M, K = a.shape
    _, N = b.shape
    BM = 1000 if M % 1000 == 0 else M
    BK = 1024 if K % 1024 == 0 else K
    grid = (M // BM, K // BK)
    return pl.pallas_call(
        functools.partial(_mm_body, act=act),
        grid=grid,
        in_specs=[pl.BlockSpec((BM, BK), lambda i, k: (i, k)),
                  pl.BlockSpec((BK, N), lambda i, k: (k, 0))],
        out_specs=pl.BlockSpec((BM, N), lambda i, k: (i, 0)),
        out_shape=jax.ShapeDtypeStruct((M, N), jnp.float32),
        compiler_params=pltpu.CompilerParams(
            dimension_semantics=("parallel", "arbitrary")),
    )(a, b)


def _prelu(x, a):
    return jnp.where(x >= 0.0, x, a * x)


def _head_body(xg_ref, xh_ref, al_ref, *refs):
    o_ref = refs[-1]
    w = [r[...] for r in refs[:-1]]

    def lb(x, ws, arow):
        w1, b1, w2, b2, w3, b3, w4, b4, w5, b5 = ws
        a2, a3, a4, a5 = (al_ref[arow, 0], al_ref[arow, 1],
                          al_ref[arow, 2], al_ref[arow, 3])
        h = jnp.dot(x, w1, preferred_element_type=jnp.float32) + b1
        h2 = jnp.dot(_prelu(h, a2), w2, preferred_element_type=jnp.float32) + b2
        h3 = jnp.dot(_prelu(h2, a3), w3, preferred_element_type=jnp.float32) + b3
        h = (h3 + h) / 2.0
        h4 = jnp.dot(_prelu(h, a4), w4, preferred_element_type=jnp.float32) + b4
        h = (h4 + h) / 2.0
        return jnp.dot(_prelu(h, a5), w5, preferred_element_type=jnp.float32) + b5

    xg = xg_ref[...]
    xh = xh_ref[...]
    o_ref[...] = lb(xg, w[:10], 0) + lb(xh, w[10:], 1)


def _head(xg, xh, p):
    def flat(blk):
        out = []
        for name in ('lin1', 'lin2', 'lin3', 'lin4', 'lin5'):
            w, b = blk[name]
            out.append(w)
            out.append(b.reshape(1, -1))
        return out

    nh, hh = p['nh_transform'], p['hh_transform']
    alphas = jnp.zeros((8, 128), jnp.float32)
    alphas = alphas.at[0, :4].set(jnp.stack([nh['a2'], nh['a3'], nh['a4'], nh['a5']]))
    alphas = alphas.at[1, :4].set(jnp.stack([hh['a2'], hh['a3'], hh['a4'], hh['a5']]))
    ws = flat(nh) + flat(hh)
    R = xg.shape[0]
    return pl.pallas_call(
        _head_body,
        out_shape=jax.ShapeDtypeStruct((R, HD), jnp.float32),
    )(xg, xh, alphas, *ws)


def _side(p, T, ni, ei, et, nt, nm, batch):
    x0 = T[ni]
    tfeat = p['type_emb'][nt]
    src, dst = ei[0], ei[1]
    nE = src.shape[0]
    deg = jax.ops.segment_sum(jnp.ones((nE,), jnp.float32), dst,
                              num_segments=N_NODES)
    dinv = (1.0 / jnp.maximum(deg, 1.0))[:, None]

    x = x0
    e = p['edge_emb'][et]
    for l, lp in enumerate(p['nh_layers']):
        S = jax.ops.segment_sum(x[src] + e, dst, num_segments=N_NODES)
        Sm = _mm(S * dinv, lp['Wm'])
        x = _mm(x + Sm + tfeat, lp['Wu'], act='relu')
        if l < 2:
            u = _mm(x, lp['We'])
            if l == 0:
                ew = p['edge_emb'] @ lp['We']
                e = jax.nn.relu(ew[et] + u[src] + u[dst])
            else:
                e = jax.nn.relu(_mm(e, lp['We']) + u[src] + u[dst])

    dh = jnp.pad(p['dHyper'], ((0, 0), (0, 120)))
    he = _mm(x0, dh)[:, :8]
    A = jax.nn.softmax(he, axis=1)
    hmax = jax.ops.segment_max(he, batch, num_segments=B_GRAPHS)
    Pm = jnp.exp(he - hmax[batch])
    Z = jax.ops.segment_sum(Pm, batch, num_segments=B_GRAPHS)
    Hs = Pm / Z[batch]
    oh = (batch[:, None] == jnp.arange(B_GRAPHS)[None, :]).astype(jnp.float32)
    Mw = (oh[:, :, None] * Hs[:, None, :]).reshape(N_NODES, 512)
    Nw = (oh[:, :, None] * A[:, None, :]).reshape(N_NODES, 512)
    MwT = Mw.T
    h = x0
    for lp in p['hh_layers']:
        efeat = _mm(MwT, h)
        up = _mm(Nw, efeat)
        h = _mm(h + up + tfeat, lp['W'], act='relu')

    idx = jnp.nonzero(nm, size=nm.shape[0] // 100)[0]
    xg = jnp.pad(x[idx], ((0, 4), (0, 0)))
    xh = jnp.pad(h[idx], ((0, 4), (0, 0)))
    return _head(xg, xh, p)[:idx.shape[0]]


def kernel(Trans_emb, params, node_index1, edge_index1, edge_type1, node_type1,
           node_mask1, batch1, node_index2, edge_index2, edge_type2, node_type2,
           node_mask2, batch2, drug_index1, drug_index2):
    p = params
    Wt, bt = p['Wt']
    T = _mm(Trans_emb, Wt) + bt
    d1 = _side(p, T, node_index1, edge_index1, edge_type1, node_type1,
               node_mask1, batch1)
    d2 = _side(p, T, node_index2, edge_index2, edge_type2, node_type2,
               node_mask2, batch2)
    return (d1, d2)


# speedup vs baseline: 2.0369x; 1.6325x over previous
import functools
import jax
import jax.numpy as jnp
from jax import lax
from jax.experimental import pallas as pl
from jax.experimental.pallas import tpu as pltpu
from jax.experimental.pallas import tpu_sc as plsc

HD = 128
N_NODES = 10000
B_GRAPHS = 64
N_EDGES_K = 320000
_C = 80
_NC, _NS = 2, 16
_NW = _NC * _NS
_PER_W = N_EDGES_K // _NW
_SLAB = 624
_TAIL = N_NODES - _SLAB * _NS


def _sc_segsum(xt, tb, ia, ib, idst):
    mesh = plsc.VectorSubcoreMesh(core_axis_name="c", subcore_axis_name="s")
    zeros = jnp.zeros((N_NODES, HD), jnp.float32)

    @functools.partial(
        pl.kernel, mesh=mesh,
        out_type=jax.ShapeDtypeStruct((_NC, N_NODES, HD), jnp.float32),
        scratch_types=[
            pltpu.VMEM((_C,), jnp.int32),
            pltpu.VMEM((_C,), jnp.int32),
            pltpu.VMEM((_C,), jnp.int32),
            pltpu.VMEM((_C, HD), jnp.float32),
            pltpu.VMEM((_C, HD), jnp.float32),
            pltpu.VMEM_SHARED((N_NODES, HD), jnp.float32),
            pltpu.SemaphoreType.DMA,
            pltpu.SemaphoreType.DMA,
        ],
    )
    def k(xt_hbm, tb_hbm, ia_hbm, ib_hbm, id_hbm, z_hbm, out_hbm,
          ia_v, ib_v, id_v, rows_v, trows_v, acc, sem_a, sem_b):
        cid = lax.axis_index("c")
        sid = lax.axis_index("s")
        wid = sid * _NC + cid
        pltpu.sync_copy(z_hbm.at[pl.ds(sid * _SLAB, _SLAB)],
                        acc.at[pl.ds(sid * _SLAB, _SLAB)])

        @pl.when(sid == _NS - 1)
        def _():
            pltpu.sync_copy(z_hbm.at[pl.ds(_SLAB * _NS, _TAIL)],
                            acc.at[pl.ds(_SLAB * _NS, _TAIL)])
        plsc.subcore_barrier()

        def body(i, carry):
            base = wid * _PER_W + i * _C
            pltpu.sync_copy(ia_hbm.at[pl.ds(base, _C)], ia_v)
            pltpu.sync_copy(ib_hbm.at[pl.ds(base, _C)], ib_v)
            pltpu.sync_copy(id_hbm.at[pl.ds(base, _C)], id_v)
            a_dma = pltpu.async_copy(xt_hbm.at[ia_v], rows_v, sem_a)
            b_dma = pltpu.async_copy(tb_hbm.at[ib_v], trows_v, sem_b)
            a_dma.wait()
            b_dma.wait()
            pltpu.sync_copy(rows_v, acc.at[id_v], add=True)
            pltpu.sync_copy(trows_v, acc.at[id_v], add=True)
            return carry

        lax.fori_loop(0, _PER_W // _C, body, 0)
        plsc.subcore_barrier()
        pltpu.sync_copy(acc.at[pl.ds(sid * _SLAB, _SLAB)],
                        out_hbm.at[cid, pl.ds(sid * _SLAB, _SLAB)])

        @pl.when(sid == _NS - 1)
        def _():
            pltpu.sync_copy(acc.at[pl.ds(_SLAB * _NS, _TAIL)],
                            out_hbm.at[cid, pl.ds(_SLAB * _NS, _TAIL)])

    return k(xt, tb, ia, ib, idst, zeros)


def _mm_body(a_ref, b_ref, o_ref, *, act):
    @pl.when(pl.program_id(1) == 0)
    def _():
        o_ref[...] = jnp.zeros_like(o_ref)

    o_ref[...] += jnp.dot(a_ref[...], b_ref[...],
                          preferred_element_type=jnp.float32)
    if act == 'relu':
        @pl.when(pl.program_id(1) == pl.num_programs(1) - 1)
        def _():
            o_ref[...] = jnp.maximum(o_ref[...], 0.0)


def _mm(a, b, act=None):
    M, K = a.shape
    _, N = b.shape
    BM = 1000 if M % 1000 == 0 else M
    BK = 1024 if K % 1024 == 0 else K
    grid = (M // BM, K // BK)
    return pl.pallas_call(
        functools.partial(_mm_body, act=act),
        grid=grid,
        in_specs=[pl.BlockSpec((BM, BK), lambda i, k: (i, k)),
                  pl.BlockSpec((BK, N), lambda i, k: (k, 0))],
        out_specs=pl.BlockSpec((BM, N), lambda i, k: (i, 0)),
        out_shape=jax.ShapeDtypeStruct((M, N), jnp.float32),
        compiler_params=pltpu.CompilerParams(
            dimension_semantics=("parallel", "arbitrary")),
    )(a, b)


def _prelu(x, a):
    return jnp.where(x >= 0.0, x, a * x)


def _head_body(xg_ref, xh_ref, al_ref, *refs):
    o_ref = refs[-1]
    w = [r[...] for r in refs[:-1]]

    def lb(x, ws, arow):
        w1, b1, w2, b2, w3, b3, w4, b4, w5, b5 = ws
        a2, a3, a4, a5 = (al_ref[arow, 0], al_ref[arow, 1],
                          al_ref[arow, 2], al_ref[arow, 3])
        h = jnp.dot(x, w1, preferred_element_type=jnp.float32) + b1
        h2 = jnp.dot(_prelu(h, a2), w2, preferred_element_type=jnp.float32) + b2
        h3 = jnp.dot(_prelu(h2, a3), w3, preferred_element_type=jnp.float32) + b3
        h = (h3 + h) / 2.0
        h4 = jnp.dot(_prelu(h, a4), w4, preferred_element_type=jnp.float32) + b4
        h = (h4 + h) / 2.0
        return jnp.dot(_prelu(h, a5), w5, preferred_element_type=jnp.float32) + b5

    xg = xg_ref[...]
    xh = xh_ref[...]
    o_ref[...] = lb(xg, w[:10], 0) + lb(xh, w[10:], 1)


def _head(xg, xh, p):
    def flat(blk):
        out = []
        for name in ('lin1', 'lin2', 'lin3', 'lin4', 'lin5'):
            w, b = blk[name]
            out.append(w)
            out.append(b.reshape(1, -1))
        return out

    nh, hh = p['nh_transform'], p['hh_transform']
    alphas = jnp.zeros((8, 128), jnp.float32)
    alphas = alphas.at[0, :4].set(jnp.stack([nh['a2'], nh['a3'], nh['a4'], nh['a5']]))
    alphas = alphas.at[1, :4].set(jnp.stack([hh['a2'], hh['a3'], hh['a4'], hh['a5']]))
    ws = flat(nh) + flat(hh)
    R = xg.shape[0]
    return pl.pallas_call(
        _head_body,
        out_shape=jax.ShapeDtypeStruct((R, HD), jnp.float32),
    )(xg, xh, alphas, *ws)


def _side(p, T, ni, ei, et, nt, nm, batch):
    x0 = T[ni]
    tfeat = p['type_emb'][nt]
    src, dst = ei[0].astype(jnp.int32), ei[1].astype(jnp.int32)
    et = et.astype(jnp.int32)
    nE = src.shape[0]
    deg = jax.ops.segment_sum(jnp.ones((nE,), jnp.float32), dst,
                              num_segments=N_NODES)
    dinv = (1.0 / jnp.maximum(deg, 1.0))[:, None]

    iota_e = jnp.arange(nE, dtype=jnp.int32)
    x = x0
    e = None
    for l, lp in enumerate(p['nh_layers']):
        if l == 0:
            Sp = _sc_segsum(x, p['edge_emb'], src, et, dst)
        else:
            Sp = _sc_segsum(x, e, src, iota_e, dst)
        S = Sp[0] + Sp[1]
        Sm = _mm(S * dinv, lp['Wm'])
        x = _mm(x + Sm + tfeat, lp['Wu'], act='relu')
        if l < 2:
            u = _mm(x, lp['We'])
            if l == 0:
                ew = p['edge_emb'] @ lp['We']
                e = jax.nn.relu(ew[et] + u[src] + u[dst])
            else:
                e = jax.nn.relu(_mm(e, lp['We']) + u[src] + u[dst])

    dh = jnp.pad(p['dHyper'], ((0, 0), (0, 120)))
    he = _mm(x0, dh)[:, :8]
    A = jax.nn.softmax(he, axis=1)
    hmax = jax.ops.segment_max(he, batch, num_segments=B_GRAPHS)
    Pm = jnp.exp(he - hmax[batch])
    Z = jax.ops.segment_sum(Pm, batch, num_segments=B_GRAPHS)
    Hs = Pm / Z[batch]
    oh = (batch[:, None] == jnp.arange(B_GRAPHS)[None, :]).astype(jnp.float32)
    Mw = (oh[:, :, None] * Hs[:, None, :]).reshape(N_NODES, 512)
    Nw = (oh[:, :, None] * A[:, None, :]).reshape(N_NODES, 512)
    MwT = Mw.T
    h = x0
    for lp in p['hh_layers']:
        efeat = _mm(MwT, h)
        up = _mm(Nw, efeat)
        h = _mm(h + up + tfeat, lp['W'], act='relu')

    idx = jnp.nonzero(nm, size=nm.shape[0] // 100)[0]
    xg = jnp.pad(x[idx], ((0, 4), (0, 0)))
    xh = jnp.pad(h[idx], ((0, 4), (0, 0)))
    return _head(xg, xh, p)[:idx.shape[0]]


def kernel(Trans_emb, params, node_index1, edge_index1, edge_type1, node_type1,
           node_mask1, batch1, node_index2, edge_index2, edge_type2, node_type2,
           node_mask2, batch2, drug_index1, drug_index2):
    p = params
    Wt, bt = p['Wt']
    T = _mm(Trans_emb, Wt) + bt
    d1 = _side(p, T, node_index1, edge_index1, edge_type1, node_type1,
               node_mask1, batch1)
    d2 = _side(p, T, node_index2, edge_index2, edge_type2, node_type2,
               node_mask2, batch2)
    return (d1, d2)
